# baseline (speedup 1.0000x reference)
import jax
import jax.numpy as jnp
from jax.experimental import pallas as pl
from jax.experimental.pallas import tpu as pltpu


def _scale_body(scale_ref, x_ref, o_ref):
    b = pl.program_id(0)
    o_ref[...] = x_ref[...] * scale_ref[b]


def kernel(input, probs, magnitudes):
    B, C, H, W = input.shape
    scale = jnp.where(probs, magnitudes, jnp.float32(1.0))
    grid = (B, C)
    out = pl.pallas_call(
        _scale_body,
        grid_spec=pltpu.PrefetchScalarGridSpec(
            num_scalar_prefetch=1,
            grid=grid,
            in_specs=[pl.BlockSpec((1, 1, H, W), lambda b, c, s: (b, c, 0, 0))],
            out_specs=pl.BlockSpec((1, 1, H, W), lambda b, c, s: (b, c, 0, 0)),
        ),
        out_shape=jax.ShapeDtypeStruct((B, C, H, W), jnp.float32),
    )(scale, input)
    return out


# speedup vs baseline: 1.5275x; 1.5275x over previous
import jax
import jax.numpy as jnp
from jax.experimental import pallas as pl
from jax.experimental.pallas import tpu as pltpu


_SB = 2


def _scale_body(scale_ref, x_ref, o_ref):
    i = pl.program_id(0)
    for j in range(_SB):
        o_ref[j] = x_ref[j] * scale_ref[i * _SB + j]


def kernel(input, probs, magnitudes):
    B, C, H, W = input.shape
    scale = jnp.where(probs, magnitudes, jnp.float32(1.0))
    grid = (B // _SB,)
    out = pl.pallas_call(
        _scale_body,
        grid_spec=pltpu.PrefetchScalarGridSpec(
            num_scalar_prefetch=1,
            grid=grid,
            in_specs=[pl.BlockSpec((_SB, C, H, W), lambda i, s: (i, 0, 0, 0))],
            out_specs=pl.BlockSpec((_SB, C, H, W), lambda i, s: (i, 0, 0, 0)),
        ),
        out_shape=jax.ShapeDtypeStruct((B, C, H, W), jnp.float32),
    )(scale, input)
    return out


# speedup vs baseline: 1.5406x; 1.0085x over previous
import jax
import jax.numpy as jnp
from jax.experimental import pallas as pl
from jax.experimental.pallas import tpu as pltpu


_SB = 4


def _scale_body(scale_ref, x_ref, o_ref):
    i = pl.program_id(0)
    for j in range(_SB):
        o_ref[j] = x_ref[j] * scale_ref[i * _SB + j]


def kernel(input, probs, magnitudes):
    B, C, H, W = input.shape
    scale = jnp.where(probs, magnitudes, jnp.float32(1.0))
    grid = (B // _SB,)
    out = pl.pallas_call(
        _scale_body,
        grid_spec=pltpu.PrefetchScalarGridSpec(
            num_scalar_prefetch=1,
            grid=grid,
            in_specs=[pl.BlockSpec((_SB, C, H, W), lambda i, s: (i, 0, 0, 0))],
            out_specs=pl.BlockSpec((_SB, C, H, W), lambda i, s: (i, 0, 0, 0)),
        ),
        out_shape=jax.ShapeDtypeStruct((B, C, H, W), jnp.float32),
    )(scale, input)
    return out
